# Initial kernel scaffold; baseline (speedup 1.0000x reference)
import jax
import jax.numpy as jnp
from jax.experimental import pallas as pl


def kernel(h_V, h_E, E_idx, mask_V, mask_attend, W1, b1, W2, b2, W3, b3, W11, b11, W12, b12, W13, b13, W_in, b_in, W_out, b_out, n1_g, n1_b, n2_g, n2_b, n3_g, n3_b):
    raise NotImplementedError("write your pallas kernel here")


# speedup vs baseline: 8.2586x; 8.2586x over previous
import functools

import jax
import jax.numpy as jnp
from jax import lax
from jax.experimental import pallas as pl
from jax.experimental.pallas import tpu as pltpu
from jax.experimental.pallas import tpu_sc as plsc

_NC, _NS = 2, 16
_NW = _NC * _NS
_SC_C = 128


def _gelu(x):
    return 0.5 * x * (1.0 + lax.erf(x * 0.7071067811865476))


def _layer_norm(x, g, b):
    mu = jnp.mean(x, axis=-1, keepdims=True)
    xc = x - mu
    var = jnp.mean(xc * xc, axis=-1, keepdims=True)
    return xc * lax.rsqrt(var + 1e-5) * g + b


def _seg_matrix(rows, cols, k, transpose):
    if transpose:
        e = lax.broadcasted_iota(jnp.int32, (cols, rows), 1) // k
        n = lax.broadcasted_iota(jnp.int32, (cols, rows), 0)
    else:
        e = lax.broadcasted_iota(jnp.int32, (rows, cols), 0) // k
        n = lax.broadcasted_iota(jnp.int32, (rows, cols), 1)
    return (e == n).astype(jnp.float32)


def _sc_gather(table, idx3, rout):
    _, h = table.shape
    nw, nch, c = idx3.shape
    rows_per_w = nch * c

    mesh = plsc.VectorSubcoreMesh(core_axis_name="c", subcore_axis_name="s")

    @functools.partial(
        pl.kernel,
        mesh=mesh,
        out_type=jax.ShapeDtypeStruct((rout, h), jnp.float32),
        scratch_types=[
            pltpu.VMEM((nch, c), jnp.int32),
            pltpu.VMEM((c, h), jnp.float32),
            pltpu.SemaphoreType.DMA,
        ],
    )
    def gather_kernel(table_hbm, idx_hbm, out_hbm, idx_v, rows_v, sem):
        wid = lax.axis_index("s") * _NC + lax.axis_index("c")
        pltpu.sync_copy(idx_hbm.at[wid], idx_v)
        base = wid * rows_per_w

        def step(i, carry):
            pltpu.async_copy(table_hbm.at[idx_v.at[i]], rows_v, sem).wait()
            pltpu.sync_copy(rows_v, out_hbm.at[pl.ds(base + i * c, c)])
            return carry

        lax.fori_loop(0, nch, step, 0)

    return gather_kernel(table, idx3)


def _pre_body(hv_ref, w1v_ref, w1g_ref, b1_ref, a1_ref, g1_ref):
    hv = hv_ref[0]
    a1_ref[0] = jnp.dot(hv, w1v_ref[...],
                        preferred_element_type=jnp.float32) + b1_ref[...]
    g1_ref[0] = jnp.dot(hv, w1g_ref[...], preferred_element_type=jnp.float32)


def _main1_body(t, k, hE_ref, g1_ref, a1_ref, hv_ref, ma_ref, mv_ref,
                w1e_ref, w2_ref, b2_ref, w3_ref, b3_ref,
                wi_ref, bi_ref, wo_ref, bo_ref,
                n1g_ref, n1b_ref, n2g_ref, n2b_ref,
                w11v_ref, b11_ref, w11g_ref,
                hv2_ref, a2_ref, g2_ref):
    f32 = jnp.float32
    e = t * k
    hE = hE_ref[0]
    R = _seg_matrix(e, t, k, transpose=False)
    x = jnp.dot(hE, w1e_ref[...], preferred_element_type=f32)
    x = x + g1_ref[0]
    x = x + jnp.dot(R, a1_ref[0], preferred_element_type=f32)
    x = _gelu(x)
    y = _gelu(jnp.dot(x, w2_ref[...], preferred_element_type=f32)
              + b2_ref[...])
    m = jnp.dot(y, w3_ref[...], preferred_element_type=f32) + b3_ref[...]
    m = m * ma_ref[0, 0]
    Rt = _seg_matrix(e, t, k, transpose=True)
    dh = jnp.dot(Rt, m, preferred_element_type=f32) * (1.0 / 30.0)
    v = _layer_norm(hv_ref[0] + dh, n1g_ref[...], n1b_ref[...])
    f = jnp.dot(_gelu(jnp.dot(v, wi_ref[...], preferred_element_type=f32)
                      + bi_ref[...]),
                wo_ref[...], preferred_element_type=f32) + bo_ref[...]
    v2 = _layer_norm(v + f, n2g_ref[...], n2b_ref[...])
    v2 = v2 * mv_ref[0, 0]
    hv2_ref[0] = v2
    a2_ref[0] = jnp.dot(v2, w11v_ref[...],
                        preferred_element_type=f32) + b11_ref[...]
    g2_ref[0] = jnp.dot(v2, w11g_ref[...], preferred_element_type=f32)


def _main2_body(t, k, hE_ref, g2_ref, a2_ref,
                w11e_ref, w12_ref, b12_ref, w13_ref, b13_ref,
                n3g_ref, n3b_ref, out_ref):
    f32 = jnp.float32
    e = t * k
    hE = hE_ref[0]
    R = _seg_matrix(e, t, k, transpose=False)
    x = jnp.dot(hE, w11e_ref[...], preferred_element_type=f32)
    x = x + g2_ref[0]
    x = _gelu(x + jnp.dot(R, a2_ref[0], preferred_element_type=f32))
    y = _gelu(jnp.dot(x, w12_ref[...], preferred_element_type=f32)
              + b12_ref[...])
    m = jnp.dot(y, w13_ref[...], preferred_element_type=f32) + b13_ref[...]
    out_ref[0] = _layer_norm(hE + m, n3g_ref[...], n3b_ref[...])


def kernel(h_V, h_E, E_idx, mask_V, mask_attend,
           W1, b1, W2, b2, W3, b3, W11, b11, W12, b12, W13, b13,
           W_in, b_in, W_out, b_out,
           n1_g, n1_b, n2_g, n2_b, n3_g, n3_b):
    B, N, H = h_V.shape
    K = E_idx.shape[-1]
    H4 = W_in.shape[1]
    T = 128
    E = T * K
    NB = N // T
    f32 = jnp.float32

    W1v, W1e, W1g = W1[:H], W1[H:2 * H], W1[2 * H:]
    W11v, W11e, W11g = W11[:H], W11[H:2 * H], W11[2 * H:]
    r1 = lambda v: v.reshape(1, -1)

    offs = (jnp.arange(B, dtype=jnp.int32) * N)[:, None, None]
    nchunks = (B * N * K) // (_NW * _SC_C)
    gidx = (E_idx.astype(jnp.int32) + offs).reshape(_NW, nchunks, _SC_C)

    wspec = lambda shape: pl.BlockSpec(shape, lambda b, n: (0, 0))

    a1, g1 = pl.pallas_call(
        _pre_body,
        grid=(B,),
        in_specs=[
            pl.BlockSpec((1, N, H), lambda b: (b, 0, 0)),
            pl.BlockSpec((H, H), lambda b: (0, 0)),
            pl.BlockSpec((H, H), lambda b: (0, 0)),
            pl.BlockSpec((1, H), lambda b: (0, 0)),
        ],
        out_specs=[pl.BlockSpec((1, N, H), lambda b: (b, 0, 0))] * 2,
        out_shape=[jax.ShapeDtypeStruct((B, N, H), f32)] * 2,
    )(h_V, W1v, W1g, r1(b1))

    G1 = _sc_gather(g1.reshape(B * N, H), gidx, B * N * K)
    G1 = G1.reshape(B, N * K, H)

    hE2 = h_E.reshape(B, N * K, H)
    ma = mask_attend.reshape(B, NB, E, 1)
    mv = mask_V.reshape(B, NB, T, 1)

    espec = pl.BlockSpec((1, E, H), lambda b, n: (b, n, 0))
    tspec = pl.BlockSpec((1, T, H), lambda b, n: (b, n, 0))
    hV2, a2, g2 = pl.pallas_call(
        functools.partial(_main1_body, T, K),
        grid=(B, NB),
        in_specs=[
            espec, espec, tspec, tspec,
            pl.BlockSpec((1, 1, E, 1), lambda b, n: (b, n, 0, 0)),
            pl.BlockSpec((1, 1, T, 1), lambda b, n: (b, n, 0, 0)),
            wspec((H, H)), wspec((H, H)), wspec((1, H)),
            wspec((H, H)), wspec((1, H)),
            wspec((H, H4)), wspec((1, H4)), wspec((H4, H)), wspec((1, H)),
            wspec((1, H)), wspec((1, H)), wspec((1, H)), wspec((1, H)),
            wspec((H, H)), wspec((1, H)), wspec((H, H)),
        ],
        out_specs=[tspec] * 3,
        out_shape=[jax.ShapeDtypeStruct((B, N, H), f32)] * 3,
    )(hE2, G1, a1, h_V, ma, mv,
      W1e, W2, r1(b2), W3, r1(b3),
      W_in, r1(b_in), W_out, r1(b_out),
      r1(n1_g), r1(n1_b), r1(n2_g), r1(n2_b),
      W11v, r1(b11), W11g)

    G2 = _sc_gather(g2.reshape(B * N, H), gidx, B * N * K)
    G2 = G2.reshape(B, N * K, H)

    hEo = pl.pallas_call(
        functools.partial(_main2_body, T, K),
        grid=(B, NB),
        in_specs=[
            espec, espec, tspec,
            wspec((H, H)), wspec((H, H)), wspec((1, H)),
            wspec((H, H)), wspec((1, H)),
            wspec((1, H)), wspec((1, H)),
        ],
        out_specs=espec,
        out_shape=jax.ShapeDtypeStruct((B, N * K, H), f32),
    )(hE2, G2, a2, W11e, W12, r1(b12), W13, r1(b13), r1(n3_g), r1(n3_b))

    return hV2, hEo.reshape(B, N, K, H)
